# baseline (speedup 1.0000x reference)
import functools

import jax
import jax.numpy as jnp
from jax import lax
from jax.experimental import pallas as pl
from jax.experimental.pallas import tpu as pltpu
from jax.experimental.pallas import tpu_sc as plsc

N = 10000
E = 320000
D = 128
G = 32
C = 3

NC = 2
NS = 16
NW = NC * NS
NP = 10240
EPW = E // NW
K = 16
NCHUNK = EPW // K
RD = 5
IM = 2 * RD
RPT = NP // NS
ZREP = RPT // K


def _sc_edge_agg(h, idx4, edge_attr):
    mesh = plsc.VectorSubcoreMesh(core_axis_name="c", subcore_axis_name="s")

    @functools.partial(
        pl.kernel,
        out_type=jax.ShapeDtypeStruct((NC, NP, D), jnp.float32),
        mesh=mesh,
        scratch_types=[
            pltpu.VMEM((IM, 2, K), jnp.int32),
            pltpu.VMEM((RD, K, D), jnp.float32),
            pltpu.VMEM((RD, K, D), jnp.float32),
            pltpu.VMEM((RD, K, D), jnp.float32),
            pltpu.VMEM_SHARED((NP, D), jnp.float32),
            pltpu.SemaphoreType.DMA,
            pltpu.SemaphoreType.DMA,
            pltpu.SemaphoreType.DMA,
            pltpu.SemaphoreType.DMA,
            pltpu.SemaphoreType.DMA,
            pltpu.SemaphoreType.DMA,
            pltpu.SemaphoreType.DMA,
            pltpu.SemaphoreType.DMA,
            pltpu.SemaphoreType.DMA,
            pltpu.SemaphoreType.DMA,
            pltpu.SemaphoreType.DMA,
            pltpu.SemaphoreType.DMA,
            pltpu.SemaphoreType.DMA,
            pltpu.SemaphoreType.DMA,
            pltpu.SemaphoreType.DMA,
            pltpu.SemaphoreType.DMA,
        ],
    )
    def k(h_hbm, idx_hbm, ea_hbm, out_hbm,
          idx_v, gbuf, ebuf, mbuf, acc_sh,
          sg0, sg1, sg2, sg3, sg4, se0, se1, se2, se3, se4,
          ss0, ss1, ss2, ss3, ss4, si):
        cid = lax.axis_index("c")
        sid = lax.axis_index("s")
        wid = sid * NC + cid
        ebase = wid * EPW

        sg = (sg0, sg1, sg2, sg3, sg4)
        se = (se0, se1, se2, se3, se4)
        ss = (ss0, ss1, ss2, ss3, ss4)

        pltpu.sync_copy(idx_hbm.at[wid, pl.ds(0, IM)], idx_v)

        for b in range(RD):
            pltpu.async_copy(h_hbm.at[idx_v.at[b, 0]], gbuf.at[b], sg[b])
            pltpu.async_copy(ea_hbm.at[pl.ds(ebase + b * K, K)],
                             ebuf.at[b], se[b])

        zeros16 = jnp.zeros((16,), jnp.float32)

        @plsc.parallel_loop(0, K)
        def _zero(i):
            for t in range(D // 16):
                mbuf[0, i, pl.ds(t * 16, 16)] = zeros16

        for r in range(ZREP):
            pltpu.sync_copy(mbuf.at[0], acc_sh.at[pl.ds(sid * RPT + r * K, K)])

        plsc.subcore_barrier()

        def process(j, b):
            q = lax.rem(j, IM)
            qn = lax.rem(j + RD, IM)

            @pl.when(j >= RD)
            def _():
                pltpu.make_async_copy(mbuf.at[b],
                                      acc_sh.at[idx_v.at[qn, 1]],
                                      ss[b]).wait()

            @pl.when(jnp.logical_and(j >= RD, j + RD < NCHUNK))
            def _():
                pltpu.async_copy(idx_hbm.at[wid, j + RD], idx_v.at[qn], si)

            pltpu.make_async_copy(h_hbm.at[idx_v.at[q, 0]], gbuf.at[b],
                                  sg[b]).wait()
            pltpu.make_async_copy(ea_hbm.at[pl.ds(ebase + j * K, K)],
                                  ebuf.at[b], se[b]).wait()

            @plsc.parallel_loop(0, K, unroll=4)
            def _compute(i):
                for t in range(D // 16):
                    sl = pl.ds(t * 16, 16)
                    mbuf[b, i, sl] = jnp.maximum(gbuf[b, i, sl]
                                                 + ebuf[b, i, sl], 0.0)

            pltpu.async_copy(mbuf.at[b], acc_sh.at[idx_v.at[q, 1]], ss[b],
                             add=True)

            @pl.when(j + RD < NCHUNK)
            def _():
                @pl.when(j >= RD)
                def _():
                    pltpu.make_async_copy(idx_hbm.at[wid, j + RD],
                                          idx_v.at[qn], si).wait()

                pltpu.async_copy(h_hbm.at[idx_v.at[qn, 0]], gbuf.at[b],
                                 sg[b])
                pltpu.async_copy(ea_hbm.at[pl.ds(ebase + (j + RD) * K, K)],
                                 ebuf.at[b], se[b])

        def group(i, _):
            for o in range(RD):
                process(RD * i + o, o)
            return 0

        lax.fori_loop(0, NCHUNK // RD, group, 0)

        for l in range(NCHUNK - RD, NCHUNK):
            pltpu.make_async_copy(mbuf.at[l % RD],
                                  acc_sh.at[idx_v.at[l % IM, 1]],
                                  ss[l % RD]).wait()

        plsc.subcore_barrier()

        pltpu.sync_copy(acc_sh.at[pl.ds(sid * RPT, RPT)],
                        out_hbm.at[cid, pl.ds(sid * RPT, RPT)])

    return k(h, idx4, edge_attr)


_BM = 2000


def _mlp_body(h_ref, a0_ref, a1_ref, w1_ref, b1_ref, w2_ref, b2_ref, o_ref):
    t = h_ref[...] + a0_ref[0] + a1_ref[0]
    t = jnp.maximum(
        jnp.dot(t, w1_ref[...], preferred_element_type=jnp.float32)
        + b1_ref[...], 0.0)
    t = (jnp.dot(t, w2_ref[...], preferred_element_type=jnp.float32)
         + b2_ref[...])
    o_ref[...] = jnp.maximum(t, 0.0)


def _tc_mlp(h, agg, w1, b1, w2, b2):
    grid = (N // _BM,)
    return pl.pallas_call(
        _mlp_body,
        grid=grid,
        in_specs=[
            pl.BlockSpec((_BM, D), lambda i: (i, 0)),
            pl.BlockSpec((1, _BM, D), lambda i: (0, i, 0)),
            pl.BlockSpec((1, _BM, D), lambda i: (1, i, 0)),
            pl.BlockSpec((D, D), lambda i: (0, 0)),
            pl.BlockSpec((1, D), lambda i: (0, 0)),
            pl.BlockSpec((D, D), lambda i: (0, 0)),
            pl.BlockSpec((1, D), lambda i: (0, 0)),
        ],
        out_specs=pl.BlockSpec((_BM, D), lambda i: (i, 0)),
        out_shape=jax.ShapeDtypeStruct((N, D), jnp.float32),
    )(h, agg, agg, w1, b1.reshape(1, D), w2, b2.reshape(1, D))


def _pool_body(h_ref, b_ref, wc_ref, bc_ref, o_ref, sums, counts):
    i = pl.program_id(0)

    @pl.when(i == 0)
    def _init():
        sums[...] = jnp.zeros_like(sums)
        counts[...] = jnp.zeros_like(counts)

    gids = lax.broadcasted_iota(jnp.int32, (_BM, G), 1)
    onehot = (b_ref[...] == gids).astype(jnp.float32)
    sums[...] += lax.dot_general(onehot, h_ref[...],
                                 (((0,), (0,)), ((), ())),
                                 preferred_element_type=jnp.float32)
    counts[...] += lax.dot_general(onehot, jnp.ones((_BM, 1), jnp.float32),
                                   (((0,), (0,)), ((), ())),
                                   preferred_element_type=jnp.float32)

    @pl.when(i == N // _BM - 1)
    def _fin():
        pooled = sums[...] / jnp.maximum(counts[...], 1.0)
        o_ref[...] = (jnp.dot(pooled, wc_ref[...],
                              preferred_element_type=jnp.float32)
                      + bc_ref[...])


def _tc_pool(h, batch2, wc, bc):
    return pl.pallas_call(
        _pool_body,
        grid=(N // _BM,),
        in_specs=[
            pl.BlockSpec((_BM, D), lambda i: (i, 0)),
            pl.BlockSpec((_BM, 1), lambda i: (i, 0)),
            pl.BlockSpec((D, C), lambda i: (0, 0)),
            pl.BlockSpec((1, C), lambda i: (0, 0)),
        ],
        out_specs=pl.BlockSpec((G, C), lambda i: (0, 0)),
        out_shape=jax.ShapeDtypeStruct((G, C), jnp.float32),
        scratch_shapes=[
            pltpu.VMEM((G, D), jnp.float32),
            pltpu.VMEM((G, 1), jnp.float32),
        ],
        compiler_params=pltpu.CompilerParams(
            dimension_semantics=("arbitrary",)),
    )(h, batch2, wc, bc.reshape(1, C))


def kernel(x, edge_index, edge_attr, batch,
           w1_0, b1_0, w2_0, b2_0,
           w1_1, b1_1, w2_1, b2_1,
           w1_2, b1_2, w2_2, b2_2,
           w1_3, b1_3, w2_3, b2_3,
           wc, bc):
    idx4 = edge_index.reshape(2, NW, NCHUNK, K).transpose(1, 2, 0, 3)
    params = [(w1_0, b1_0, w2_0, b2_0), (w1_1, b1_1, w2_1, b2_1),
              (w1_2, b1_2, w2_2, b2_2), (w1_3, b1_3, w2_3, b2_3)]
    h = x
    for (w1, b1, w2, b2) in params:
        agg = _sc_edge_agg(h, idx4, edge_attr)
        h = _tc_mlp(h, agg, w1, b1, w2, b2)
    return _tc_pool(h, batch.reshape(N, 1), wc, bc)
